# baseline (speedup 1.0000x reference)
import functools

import jax
import jax.numpy as jnp
from jax import lax
from jax.experimental import pallas as pl
from jax.experimental.pallas import tpu as pltpu
from jax.experimental.pallas import tpu_sc as plsc

_L = 16


def kernel(heads, rels, entity_table, rel_table):
    B = heads.shape[0]
    E, D = entity_table.shape
    R = rel_table.shape[0]

    info = plsc.get_sparse_core_info()
    NW = info.num_cores * info.num_subcores
    b_w = B // NW
    n_grp = b_w // _L
    assert b_w * NW == B and n_grp * _L == b_w

    et_lin = entity_table.reshape(E // 4, 128)
    rt_lin = rel_table.reshape(R // 4, 128)

    mesh = plsc.VectorSubcoreMesh(core_axis_name="c", subcore_axis_name="s")

    @functools.partial(
        pl.kernel,
        mesh=mesh,
        compiler_params=pltpu.CompilerParams(needs_layout_passes=False),
        out_type=(
            jax.ShapeDtypeStruct((B * D,), jnp.float32),
            jax.ShapeDtypeStruct((B * D,), jnp.float32),
        ),
        scratch_types=[
            pltpu.VMEM((b_w,), jnp.int32),
            pltpu.VMEM((b_w,), jnp.int32),
            pltpu.VMEM((b_w * D,), jnp.float32),
            pltpu.VMEM((b_w * D,), jnp.float32),
            pltpu.SemaphoreType.DMA,
            pltpu.SemaphoreType.DMA,
            pltpu.SemaphoreType.DMA,
        ],
    )
    def _gather2(heads_hbm, rels_hbm, et_hbm, rt_hbm,
                 out_h_hbm, out_r_hbm,
                 hidx_v, ridx_v, houtv, routv, sem_i, sem_h, sem_r):
        wid = lax.axis_index("s") * info.num_cores + lax.axis_index("c")
        base = wid * b_w

        ci = pltpu.async_copy(heads_hbm.at[pl.ds(base, b_w)], hidx_v, sem_i)
        cr = pltpu.async_copy(rels_hbm.at[pl.ds(base, b_w)], ridx_v, sem_i)
        ci.wait()
        cr.wait()

        def _grp(g, _):
            hv = hidx_v[pl.ds(g * _L, _L)]
            rv = ridx_v[pl.ds(g * _L, _L)]
            ob = g * _L * D
            for lane in range(_L):
                h = hv[lane]
                pltpu.async_copy(
                    et_hbm.at[h >> 2, pl.ds((h & 3) * D, D)],
                    houtv.at[pl.ds(ob + lane * D, D)], sem_h)
                r = rv[lane]
                pltpu.async_copy(
                    rt_hbm.at[r >> 2, pl.ds((r & 3) * D, D)],
                    routv.at[pl.ds(ob + lane * D, D)], sem_r)
            return 0
        lax.fori_loop(0, n_grp, _grp, 0)

        pltpu.make_async_copy(
            out_h_hbm.at[pl.ds(base * D, b_w * D)], houtv, sem_h).wait()
        pltpu.make_async_copy(
            out_r_hbm.at[pl.ds(base * D, b_w * D)], routv, sem_r).wait()

        pltpu.sync_copy(houtv, out_h_hbm.at[pl.ds(base * D, b_w * D)])
        pltpu.sync_copy(routv, out_r_hbm.at[pl.ds(base * D, b_w * D)])

    out_h, out_r = _gather2(heads, rels, et_lin, rt_lin)
    return (out_h.reshape(B, D), out_r.reshape(B, D))
